# Initial kernel scaffold; baseline (speedup 1.0000x reference)
import jax
import jax.numpy as jnp
from jax.experimental import pallas as pl
from jax.experimental.pallas import tpu as pltpu

N = 100000
C = 128
S = 512
BLK = 2000
NBLK = N // BLK

_NEG = jnp.float32(-1e30)


def _stats_body(xb, wb, bb, scores_out, m_out, l_out, m_ref, l_ref):
    j = pl.program_id(0)

    @pl.when(j == 0)
    def _():
        m_ref[...] = jnp.full_like(m_ref, _NEG)
        l_ref[...] = jnp.zeros_like(l_ref)

    x = xb[...]
    w = wb[...]
    s = jnp.dot(x, w, preferred_element_type=jnp.float32)[:, 0]
    scores_out[0, 0, :] = s

    b = bb[0, 0, :]
    seg = jax.lax.broadcasted_iota(jnp.int32, (BLK, S), 1)
    onehot = b[:, None] == seg
    smask = jnp.where(onehot, s[:, None], _NEG)
    bmax = jnp.max(smask, axis=0, keepdims=True)
    m_old = m_ref[...]
    m_new = jnp.maximum(m_old, bmax)
    e = jnp.where(onehot, jnp.exp(smask - m_new), 0.0)
    bsum = jnp.sum(e, axis=0, keepdims=True)
    l_ref[...] = l_ref[...] * jnp.exp(m_old - m_new) + bsum
    m_ref[...] = m_new

    @pl.when(j == NBLK - 1)
    def _():
        m_out[...] = m_ref[...]
        l_out[...] = l_ref[...]


def _readout_body(xb, bb, sc, m_in, l_in, attn_out, ro_out, acc_ref):
    j = pl.program_id(0)

    @pl.when(j == 0)
    def _():
        acc_ref[...] = jnp.zeros_like(acc_ref)

    x = xb[...]
    s = sc[0, 0, :]
    b = bb[0, 0, :]
    seg = jax.lax.broadcasted_iota(jnp.int32, (BLK, S), 1)
    onehot = (b[:, None] == seg).astype(jnp.float32)
    m_b = jnp.dot(onehot, m_in[0][:, None],
                  preferred_element_type=jnp.float32)[:, 0]
    l_b = jnp.dot(onehot, l_in[0][:, None],
                  preferred_element_type=jnp.float32)[:, 0]
    attn = jnp.exp(s - m_b) / l_b
    attn_out[0, 0, :] = attn
    wm = onehot * attn[:, None]
    acc_ref[...] += jax.lax.dot_general(
        wm, x, (((0,), (0,)), ((), ())),
        preferred_element_type=jnp.float32)

    @pl.when(j == NBLK - 1)
    def _():
        ro_out[...] = acc_ref[...]


def kernel(x, W_a, batch):
    b3 = batch.astype(jnp.int32).reshape(NBLK, 1, BLK)

    scores, m, l = pl.pallas_call(
        _stats_body,
        grid=(NBLK,),
        in_specs=[
            pl.BlockSpec((BLK, C), lambda j: (j, 0)),
            pl.BlockSpec((C, 1), lambda j: (0, 0)),
            pl.BlockSpec((1, 1, BLK), lambda j: (j, 0, 0)),
        ],
        out_specs=[
            pl.BlockSpec((1, 1, BLK), lambda j: (j, 0, 0)),
            pl.BlockSpec((1, S), lambda j: (0, 0)),
            pl.BlockSpec((1, S), lambda j: (0, 0)),
        ],
        out_shape=[
            jax.ShapeDtypeStruct((NBLK, 1, BLK), jnp.float32),
            jax.ShapeDtypeStruct((1, S), jnp.float32),
            jax.ShapeDtypeStruct((1, S), jnp.float32),
        ],
        scratch_shapes=[
            pltpu.VMEM((1, S), jnp.float32),
            pltpu.VMEM((1, S), jnp.float32),
        ],
        compiler_params=pltpu.CompilerParams(
            dimension_semantics=("arbitrary",)),
    )(x, W_a, b3)

    attn, ro = pl.pallas_call(
        _readout_body,
        grid=(NBLK,),
        in_specs=[
            pl.BlockSpec((BLK, C), lambda j: (j, 0)),
            pl.BlockSpec((1, 1, BLK), lambda j: (j, 0, 0)),
            pl.BlockSpec((1, 1, BLK), lambda j: (j, 0, 0)),
            pl.BlockSpec((1, S), lambda j: (0, 0)),
            pl.BlockSpec((1, S), lambda j: (0, 0)),
        ],
        out_specs=[
            pl.BlockSpec((1, 1, BLK), lambda j: (j, 0, 0)),
            pl.BlockSpec((S, C), lambda j: (0, 0)),
        ],
        out_shape=[
            jax.ShapeDtypeStruct((NBLK, 1, BLK), jnp.float32),
            jax.ShapeDtypeStruct((S, C), jnp.float32),
        ],
        scratch_shapes=[
            pltpu.VMEM((S, C), jnp.float32),
        ],
        compiler_params=pltpu.CompilerParams(
            dimension_semantics=("arbitrary",)),
    )(x, b3, scores, m, l)

    return ro, attn.reshape(N)


# speedup vs baseline: 2.1419x; 2.1419x over previous
import jax
import jax.numpy as jnp
from jax.experimental import pallas as pl
from jax.experimental.pallas import tpu as pltpu

N = 100000
C = 128
S = 512
BLK = 2000
NBLK = N // BLK

_NEG = -1e30


def _stats_body(xb, wb, bb, scores_out, m_out, l_out, m_ref, l_ref):
    j = pl.program_id(0)

    @pl.when(j == 0)
    def _():
        m_ref[...] = jnp.full_like(m_ref, _NEG)
        l_ref[...] = jnp.zeros_like(l_ref)

    x = xb[...]
    w = wb[...]
    s = jnp.dot(x, w, preferred_element_type=jnp.float32,
                precision=jax.lax.Precision.HIGHEST)[:, 0]
    scores_out[0, 0, :] = s

    b = bb[0, 0, :]
    seg = jax.lax.broadcasted_iota(jnp.int32, (BLK, S), 1)
    onehot = b[:, None] == seg
    smask = jnp.where(onehot, s[:, None], _NEG)
    bmax = jnp.max(smask, axis=0, keepdims=True)
    m_old = m_ref[...]
    m_new = jnp.maximum(m_old, bmax)
    e = jnp.where(onehot, jnp.exp(smask - m_new), 0.0)
    bsum = jnp.sum(e, axis=0, keepdims=True)
    l_ref[...] = l_ref[...] * jnp.exp(m_old - m_new) + bsum
    m_ref[...] = m_new

    @pl.when(j == NBLK - 1)
    def _():
        m_out[...] = m_ref[...]
        l_out[...] = l_ref[...]


def _readout_body(xb, bb, sc, m_in, l_in, attn_out, ro_out, acc_ref):
    j = pl.program_id(0)

    @pl.when(j == 0)
    def _():
        acc_ref[...] = jnp.zeros_like(acc_ref)

    x = xb[...]
    s = sc[0, 0, :]
    b = bb[0, 0, :]
    seg = jax.lax.broadcasted_iota(jnp.int32, (BLK, S), 1)
    onehot = (b[:, None] == seg).astype(jnp.float32)
    m_b = jnp.dot(onehot, m_in[0][:, None],
                  preferred_element_type=jnp.float32,
                  precision=jax.lax.Precision.HIGHEST)[:, 0]
    l_b = jnp.dot(onehot, l_in[0][:, None],
                  preferred_element_type=jnp.float32,
                  precision=jax.lax.Precision.HIGHEST)[:, 0]
    attn = jnp.exp(s - m_b) / l_b
    attn_out[0, 0, :] = attn
    wm = onehot * attn[:, None]
    acc_ref[...] += jax.lax.dot_general(
        wm, x, (((0,), (0,)), ((), ())),
        preferred_element_type=jnp.float32,
        precision=jax.lax.Precision.HIGHEST)

    @pl.when(j == NBLK - 1)
    def _():
        ro_out[...] = acc_ref[...]


def kernel(x, W_a, batch):
    b3 = batch.astype(jnp.int32).reshape(NBLK, 1, BLK)

    scores, m, l = pl.pallas_call(
        _stats_body,
        grid=(NBLK,),
        in_specs=[
            pl.BlockSpec((BLK, C), lambda j: (j, 0)),
            pl.BlockSpec((C, 1), lambda j: (0, 0)),
            pl.BlockSpec((1, 1, BLK), lambda j: (j, 0, 0)),
        ],
        out_specs=[
            pl.BlockSpec((1, 1, BLK), lambda j: (j, 0, 0)),
            pl.BlockSpec((1, S), lambda j: (0, 0)),
            pl.BlockSpec((1, S), lambda j: (0, 0)),
        ],
        out_shape=[
            jax.ShapeDtypeStruct((NBLK, 1, BLK), jnp.float32),
            jax.ShapeDtypeStruct((1, S), jnp.float32),
            jax.ShapeDtypeStruct((1, S), jnp.float32),
        ],
        scratch_shapes=[
            pltpu.VMEM((1, S), jnp.float32),
            pltpu.VMEM((1, S), jnp.float32),
        ],
        compiler_params=pltpu.CompilerParams(
            dimension_semantics=("arbitrary",)),
    )(x, W_a, b3)

    attn, ro = pl.pallas_call(
        _readout_body,
        grid=(NBLK,),
        in_specs=[
            pl.BlockSpec((BLK, C), lambda j: (j, 0)),
            pl.BlockSpec((1, 1, BLK), lambda j: (j, 0, 0)),
            pl.BlockSpec((1, 1, BLK), lambda j: (j, 0, 0)),
            pl.BlockSpec((1, S), lambda j: (0, 0)),
            pl.BlockSpec((1, S), lambda j: (0, 0)),
        ],
        out_specs=[
            pl.BlockSpec((1, 1, BLK), lambda j: (j, 0, 0)),
            pl.BlockSpec((S, C), lambda j: (0, 0)),
        ],
        out_shape=[
            jax.ShapeDtypeStruct((NBLK, 1, BLK), jnp.float32),
            jax.ShapeDtypeStruct((S, C), jnp.float32),
        ],
        scratch_shapes=[
            pltpu.VMEM((S, C), jnp.float32),
        ],
        compiler_params=pltpu.CompilerParams(
            dimension_semantics=("arbitrary",)),
    )(x, b3, scores, m, l)

    return ro, attn.reshape(N)


# speedup vs baseline: 2.5101x; 1.1719x over previous
import jax
import jax.numpy as jnp
from jax.experimental import pallas as pl
from jax.experimental.pallas import tpu as pltpu

N = 100000
C = 128
S = 512
BLK = 2000
NBLK = N // BLK

_NEG = -1e30


def _stats_body(xb, wb, bb, scores_out, m_out, l_out, m_ref, l_ref):
    j = pl.program_id(0)

    @pl.when(j == 0)
    def _():
        m_ref[...] = jnp.full_like(m_ref, _NEG)
        l_ref[...] = jnp.zeros_like(l_ref)

    x = xb[...]
    w = wb[...]
    s = jnp.dot(x, w, preferred_element_type=jnp.float32,
                precision=jax.lax.Precision.HIGHEST)[:, 0]
    scores_out[0, 0, :] = s

    b = bb[0, 0, :]
    seg = jax.lax.broadcasted_iota(jnp.int32, (BLK, S), 1)
    onehot = b[:, None] == seg
    smask = jnp.where(onehot, s[:, None], _NEG)
    bmax = jnp.max(smask, axis=0, keepdims=True)
    m_old = m_ref[...]
    m_new = jnp.maximum(m_old, bmax)
    e = jnp.where(onehot, jnp.exp(smask - m_new), 0.0)
    bsum = jnp.sum(e, axis=0, keepdims=True)
    l_ref[...] = l_ref[...] * jnp.exp(m_old - m_new) + bsum
    m_ref[...] = m_new

    @pl.when(j == NBLK - 1)
    def _():
        m_out[...] = m_ref[...]
        l_out[...] = l_ref[...]


def _readout_body(xb, bb, sc, m_in, l_in, attn_out, ro_out, acc_ref):
    j = pl.program_id(0)

    @pl.when(j == 0)
    def _():
        acc_ref[...] = jnp.zeros_like(acc_ref)

    x = xb[...]
    s = sc[0, 0, :]
    b = bb[0, 0, :]
    seg = jax.lax.broadcasted_iota(jnp.int32, (BLK, S), 1)
    onehot = (b[:, None] == seg).astype(jnp.float32)
    m_b = jnp.dot(onehot, m_in[0][:, None],
                  preferred_element_type=jnp.float32,
                  precision=jax.lax.Precision.HIGHEST)[:, 0]
    l_b = jnp.dot(onehot, l_in[0][:, None],
                  preferred_element_type=jnp.float32,
                  precision=jax.lax.Precision.HIGHEST)[:, 0]
    attn = jnp.exp(s - m_b) / l_b
    attn_out[0, 0, :] = attn
    wm = onehot * attn[:, None]
    acc_ref[...] += jax.lax.dot_general(
        wm, x, (((0,), (0,)), ((), ())),
        preferred_element_type=jnp.float32)

    @pl.when(j == NBLK - 1)
    def _():
        ro_out[...] = acc_ref[...]


def kernel(x, W_a, batch):
    b3 = batch.astype(jnp.int32).reshape(NBLK, 1, BLK)

    scores, m, l = pl.pallas_call(
        _stats_body,
        grid=(NBLK,),
        in_specs=[
            pl.BlockSpec((BLK, C), lambda j: (j, 0)),
            pl.BlockSpec((C, 1), lambda j: (0, 0)),
            pl.BlockSpec((1, 1, BLK), lambda j: (j, 0, 0)),
        ],
        out_specs=[
            pl.BlockSpec((1, 1, BLK), lambda j: (j, 0, 0)),
            pl.BlockSpec((1, S), lambda j: (0, 0)),
            pl.BlockSpec((1, S), lambda j: (0, 0)),
        ],
        out_shape=[
            jax.ShapeDtypeStruct((NBLK, 1, BLK), jnp.float32),
            jax.ShapeDtypeStruct((1, S), jnp.float32),
            jax.ShapeDtypeStruct((1, S), jnp.float32),
        ],
        scratch_shapes=[
            pltpu.VMEM((1, S), jnp.float32),
            pltpu.VMEM((1, S), jnp.float32),
        ],
        compiler_params=pltpu.CompilerParams(
            dimension_semantics=("arbitrary",)),
    )(x, W_a, b3)

    attn, ro = pl.pallas_call(
        _readout_body,
        grid=(NBLK,),
        in_specs=[
            pl.BlockSpec((BLK, C), lambda j: (j, 0)),
            pl.BlockSpec((1, 1, BLK), lambda j: (j, 0, 0)),
            pl.BlockSpec((1, 1, BLK), lambda j: (j, 0, 0)),
            pl.BlockSpec((1, S), lambda j: (0, 0)),
            pl.BlockSpec((1, S), lambda j: (0, 0)),
        ],
        out_specs=[
            pl.BlockSpec((1, 1, BLK), lambda j: (j, 0, 0)),
            pl.BlockSpec((S, C), lambda j: (0, 0)),
        ],
        out_shape=[
            jax.ShapeDtypeStruct((NBLK, 1, BLK), jnp.float32),
            jax.ShapeDtypeStruct((S, C), jnp.float32),
        ],
        scratch_shapes=[
            pltpu.VMEM((S, C), jnp.float32),
        ],
        compiler_params=pltpu.CompilerParams(
            dimension_semantics=("arbitrary",)),
    )(x, b3, scores, m, l)

    return ro, attn.reshape(N)
